# baseline (speedup 1.0000x reference)
import functools

import jax
import jax.numpy as jnp
from jax import lax
from jax.experimental import pallas as pl
from jax.experimental.pallas import tpu as pltpu
from jax.experimental.pallas import tpu_sc as plsc

B = 16384
AE = 50
AEP = 56
NE = 20
NEP = 24
NODE_V = 1000
CTX = 512
ST = 256
OUT_D = AE + CTX + AE + ST + NE
OFF_CTX = AE
OFF_PAR = AE + CTX
OFF_ST = OFF_PAR + AE
OFF_NODE = OFF_ST + ST
CT = OUT_D // 8
RT = B // 128

NC, NS, L = 2, 16, 16
NW = NC * NS
ROWS_PER_W = B // NW
C = 32
NCHUNK = ROWS_PER_W // C
TR = C // 8
CTX_TC = CTX // 128
ST_TC = ST // 128


def _body(node_idx_hbm, act_p_hbm, st_hbm, act_q_hbm, ctx_hbm,
          rule_hbm, token_hbm, ntab_hbm, out_hbm,
          tp_v, vp_v, tq_v, vq_v, nidx_v, emb_p_v, emb_q_v, ntab_v,
          ctx_s, st_s, out_v, sem_in, sem_g, sem_n):
  wid = lax.axis_index("s") * NC + lax.axis_index("c")
  base_w = wid * ROWS_PER_W
  iota = lax.iota(jnp.int32, L)

  nt = pltpu.async_copy(ntab_hbm, ntab_v, sem_n)

  def chunk(g, carry):
    row0 = base_w + g * C
    rt0 = row0 // 128
    lane0 = row0 % 128

    i1 = pltpu.async_copy(act_p_hbm.at[pl.ds(row0, C)], tp_v, sem_in)
    i2 = pltpu.async_copy(act_p_hbm.at[pl.ds(B + row0, C)], vp_v, sem_in)
    i3 = pltpu.async_copy(act_q_hbm.at[pl.ds(row0, C)], tq_v, sem_in)
    i4 = pltpu.async_copy(act_q_hbm.at[pl.ds(B + row0, C)], vq_v, sem_in)
    i5 = pltpu.async_copy(node_idx_hbm.at[pl.ds(row0, C)], nidx_v, sem_in)
    d1 = pltpu.async_copy(ctx_hbm.at[pl.ds(row0 // 8, TR)], ctx_s, sem_in)
    d2 = pltpu.async_copy(st_hbm.at[pl.ds(row0 // 8, TR)], st_s, sem_in)
    i1.wait(); i2.wait(); i3.wait(); i4.wait(); i5.wait()

    def fetch_group(j, _):
      tp_vec = tp_v[pl.ds(j * L, L)]
      vp_vec = vp_v[pl.ds(j * L, L)]
      tq_vec = tq_v[pl.ds(j * L, L)]
      vq_vec = vq_v[pl.ds(j * L, L)]
      for l in range(L):
        i = j * L + l
        tp, vp = tp_vec[l], vp_vec[l]
        tq, vq = tq_vec[l], vq_vec[l]

        @pl.when(tp == 0)
        def _():
          pltpu.async_copy(rule_hbm.at[pl.ds(vp * AEP, AEP)],
                           emb_p_v.at[pl.ds(i * AEP, AEP)], sem_g)

        @pl.when(tp != 0)
        def _():
          pltpu.async_copy(token_hbm.at[pl.ds(vp * AEP, AEP)],
                           emb_p_v.at[pl.ds(i * AEP, AEP)], sem_g)

        @pl.when(tq == 0)
        def _():
          pltpu.async_copy(rule_hbm.at[pl.ds(vq * AEP, AEP)],
                           emb_q_v.at[pl.ds(i * AEP, AEP)], sem_g)

        @pl.when(tq != 0)
        def _():
          pltpu.async_copy(token_hbm.at[pl.ds(vq * AEP, AEP)],
                           emb_q_v.at[pl.ds(i * AEP, AEP)], sem_g)

      return 0

    lax.fori_loop(0, C // L, fetch_group, 0)

    d1.wait(); d2.wait()
    for seg_off, seg_tc, seg_s in ((OFF_CTX, CTX_TC, ctx_s),
                                   (OFF_ST, ST_TC, st_s)):
      for tc in range(seg_tc):
        for k in range(128 // L):
          c0 = seg_off + tc * 128 + k * L
          cvec = iota + c0
          cdiv = cvec // 8
          cmod = lax.rem(cvec, jnp.full((L,), 8, jnp.int32))
          for tr in range(TR):
            for sl in range(8):
              r = tr * 8 + sl
              v = seg_s[tr, tc, sl, pl.ds(k * L, L)]
              plsc.store_scatter(out_v, [cdiv, cmod, iota * 0 + r], v)

    def node_col(c, _):
      for j in range(C // L):
        rows = iota + (j * L)
        ids = nidx_v[pl.ds(j * L, L)]
        v = plsc.load_gather(ntab_v, [ids * NEP + c])
        cc = c + OFF_NODE
        out_v[cc // 8, lax.rem(cc, 8), pl.ds(j * L, L)] = v
      return 0

    lax.fori_loop(0, NE, node_col, 0)

    pltpu.make_async_copy(rule_hbm.at[pl.ds(0, C * AEP)], emb_p_v,
                          sem_g).wait()
    pltpu.make_async_copy(rule_hbm.at[pl.ds(0, C * AEP)], emb_q_v,
                          sem_g).wait()

    def place_col(c, _):
      for j in range(C // L):
        rows = iota + (j * L)
        flat = rows * AEP + c
        vp = plsc.load_gather(emb_p_v, [flat])
        out_v[c // 8, lax.rem(c, 8), pl.ds(j * L, L)] = vp
        vq = plsc.load_gather(emb_q_v, [flat])
        cq = c + OFF_PAR
        out_v[cq // 8, lax.rem(cq, 8), pl.ds(j * L, L)] = vq
      return 0

    lax.fori_loop(0, AE, place_col, 0)

    pltpu.sync_copy(out_v, out_hbm.at[:, rt0, :, pl.ds(lane0, C)])
    return carry

  nt.wait()
  lax.fori_loop(0, NCHUNK, chunk, 0)


@jax.jit
def _lstm_input(node_idx, act_p, st4, act_q, ctx4, rule_flat, token_flat,
                ntab_flat):
  mesh = plsc.VectorSubcoreMesh(core_axis_name="c", subcore_axis_name="s",
                                num_cores=NC, num_subcores=NS)
  f = functools.partial(
      pl.kernel,
      out_type=jax.ShapeDtypeStruct((CT, RT, 8, 128), jnp.float32),
      mesh=mesh,
      scratch_types=[
          pltpu.VMEM((C,), jnp.int32),
          pltpu.VMEM((C,), jnp.int32),
          pltpu.VMEM((C,), jnp.int32),
          pltpu.VMEM((C,), jnp.int32),
          pltpu.VMEM((C,), jnp.int32),
          pltpu.VMEM((C * AEP,), jnp.float32),
          pltpu.VMEM((C * AEP,), jnp.float32),
          pltpu.VMEM((NODE_V * NEP,), jnp.float32),
          pltpu.VMEM((TR, CTX_TC, 8, 128), jnp.float32),
          pltpu.VMEM((TR, ST_TC, 8, 128), jnp.float32),
          pltpu.VMEM((CT, 8, C), jnp.float32),
          pltpu.SemaphoreType.DMA,
          pltpu.SemaphoreType.DMA,
          pltpu.SemaphoreType.DMA,
      ],
      compiler_params=pltpu.CompilerParams(use_tc_tiling_on_sc=False,
                                           needs_layout_passes=False),
  )(_body)
  out4 = f(node_idx, act_p, st4, act_q, ctx4, rule_flat, token_flat,
           ntab_flat)
  return out4.transpose(0, 2, 1, 3).reshape(OUT_D, B).T


def kernel(current_node_type, previous_action, parent_state, parent_action,
           context, rule_embedding_table, token_embedding_table,
           node_embedding_table):
  act_p = previous_action.astype(jnp.int32).T.reshape(-1)
  act_q = parent_action.astype(jnp.int32).T.reshape(-1)
  ctx4 = context.reshape(B // 8, 8, CTX // 128, 128).transpose(0, 2, 1, 3)
  st4 = parent_state.reshape(B // 8, 8, ST // 128, 128).transpose(0, 2, 1, 3)
  rule_flat = jnp.pad(rule_embedding_table, ((0, 0), (0, AEP - AE))).reshape(-1)
  token_flat = jnp.pad(token_embedding_table, ((0, 0), (0, AEP - AE))).reshape(-1)
  ntab_flat = jnp.pad(node_embedding_table, ((0, 0), (0, NEP - NE))).reshape(-1)
  return _lstm_input(current_node_type.astype(jnp.int32), act_p, st4, act_q,
                     ctx4, rule_flat, token_flat, ntab_flat)


# speedup vs baseline: 1.0017x; 1.0017x over previous
import functools

import jax
import jax.numpy as jnp
from jax import lax
from jax.experimental import pallas as pl
from jax.experimental.pallas import tpu as pltpu
from jax.experimental.pallas import tpu_sc as plsc

B = 16384
AE = 50
AEP = 56
NE = 20
NEP = 24
NODE_V = 1000
CTX = 512
ST = 256
OUT_D = AE + CTX + AE + ST + NE
OFF_CTX = AE
OFF_PAR = AE + CTX
OFF_ST = OFF_PAR + AE
OFF_NODE = OFF_ST + ST
CT = OUT_D // 8
RT = B // 128

NC, NS, L = 2, 16, 16
NW = NC * NS
ROWS_PER_W = B // NW
C = 32
NCHUNK = ROWS_PER_W // C
TR = C // 8
CTX_TC = CTX // 128
ST_TC = ST // 128
ACC = 2


def _body(node_idx_hbm, act_p_hbm, st_hbm, act_q_hbm, ctx_hbm,
          rule_hbm, token_hbm, ntab_hbm, out_hbm,
          tp_v, vp_v, tq_v, vq_v, nidx_v, emb_p_v, emb_q_v, ntab_v,
          ctx_s, st_s, out_v, sem_in, sem_g, sem_n):
  wid = lax.axis_index("s") * NC + lax.axis_index("c")
  base_w = wid * ROWS_PER_W
  iota = lax.iota(jnp.int32, L)

  nt = pltpu.async_copy(ntab_hbm, ntab_v, sem_n)

  def chunk(g, carry):
    row0 = base_w + g * C
    rt0 = row0 // 128
    lane0 = row0 % 128
    acc0 = lax.rem(row0, ACC * C)

    i1 = pltpu.async_copy(act_p_hbm.at[pl.ds(row0, C)], tp_v, sem_in)
    i2 = pltpu.async_copy(act_p_hbm.at[pl.ds(B + row0, C)], vp_v, sem_in)
    i3 = pltpu.async_copy(act_q_hbm.at[pl.ds(row0, C)], tq_v, sem_in)
    i4 = pltpu.async_copy(act_q_hbm.at[pl.ds(B + row0, C)], vq_v, sem_in)
    i5 = pltpu.async_copy(node_idx_hbm.at[pl.ds(row0, C)], nidx_v, sem_in)
    d1 = pltpu.async_copy(ctx_hbm.at[pl.ds(row0 // 8, TR)], ctx_s, sem_in)
    d2 = pltpu.async_copy(st_hbm.at[pl.ds(row0 // 8, TR)], st_s, sem_in)
    i1.wait(); i2.wait(); i3.wait(); i4.wait(); i5.wait()

    def fetch_group(j, _):
      tp_vec = tp_v[pl.ds(j * L, L)]
      vp_vec = vp_v[pl.ds(j * L, L)]
      tq_vec = tq_v[pl.ds(j * L, L)]
      vq_vec = vq_v[pl.ds(j * L, L)]
      for l in range(L):
        i = j * L + l
        tp, vp = tp_vec[l], vp_vec[l]
        tq, vq = tq_vec[l], vq_vec[l]

        @pl.when(tp == 0)
        def _():
          pltpu.async_copy(rule_hbm.at[pl.ds(vp * AEP, AEP)],
                           emb_p_v.at[pl.ds(i * AEP, AEP)], sem_g)

        @pl.when(tp != 0)
        def _():
          pltpu.async_copy(token_hbm.at[pl.ds(vp * AEP, AEP)],
                           emb_p_v.at[pl.ds(i * AEP, AEP)], sem_g)

        @pl.when(tq == 0)
        def _():
          pltpu.async_copy(rule_hbm.at[pl.ds(vq * AEP, AEP)],
                           emb_q_v.at[pl.ds(i * AEP, AEP)], sem_g)

        @pl.when(tq != 0)
        def _():
          pltpu.async_copy(token_hbm.at[pl.ds(vq * AEP, AEP)],
                           emb_q_v.at[pl.ds(i * AEP, AEP)], sem_g)

      return 0

    lax.fori_loop(0, C // L, fetch_group, 0)

    d1.wait(); d2.wait()
    for seg_off, seg_tc, seg_s in ((OFF_CTX, CTX_TC, ctx_s),
                                   (OFF_ST, ST_TC, st_s)):
      for tc in range(seg_tc):
        for k in range(128 // L):
          c0 = seg_off + tc * 128 + k * L
          cvec = iota + c0
          cdiv = cvec // 8
          cmod = lax.rem(cvec, jnp.full((L,), 8, jnp.int32))
          for tr in range(TR):
            for sl in range(8):
              r = tr * 8 + sl
              v = seg_s[tr, tc, sl, pl.ds(k * L, L)]
              plsc.store_scatter(out_v, [cdiv, cmod, iota * 0 + (acc0 + r)], v)

    def node_col(c, _):
      for j in range(C // L):
        rows = iota + (j * L)
        ids = nidx_v[pl.ds(j * L, L)]
        v = plsc.load_gather(ntab_v, [ids * NEP + c])
        cc = c + OFF_NODE
        out_v[cc // 8, lax.rem(cc, 8), pl.ds(acc0 + j * L, L)] = v
      return 0

    lax.fori_loop(0, NE, node_col, 0)

    pltpu.make_async_copy(rule_hbm.at[pl.ds(0, C * AEP)], emb_p_v,
                          sem_g).wait()
    pltpu.make_async_copy(rule_hbm.at[pl.ds(0, C * AEP)], emb_q_v,
                          sem_g).wait()

    def place_col(c, _):
      for j in range(C // L):
        rows = iota + (j * L)
        flat = rows * AEP + c
        vp = plsc.load_gather(emb_p_v, [flat])
        out_v[c // 8, lax.rem(c, 8), pl.ds(acc0 + j * L, L)] = vp
        vq = plsc.load_gather(emb_q_v, [flat])
        cq = c + OFF_PAR
        out_v[cq // 8, lax.rem(cq, 8), pl.ds(acc0 + j * L, L)] = vq
      return 0

    lax.fori_loop(0, AE, place_col, 0)

    @pl.when(lax.rem(g, ACC) == ACC - 1)
    def _():
      pltpu.sync_copy(out_v,
                      out_hbm.at[:, rt0, :, pl.ds(lane0 - (ACC - 1) * C,
                                                  ACC * C)])

    return carry

  nt.wait()
  lax.fori_loop(0, NCHUNK, chunk, 0)


@jax.jit
def _lstm_input(node_idx, act_p, st4, act_q, ctx4, rule_flat, token_flat,
                ntab_flat):
  mesh = plsc.VectorSubcoreMesh(core_axis_name="c", subcore_axis_name="s",
                                num_cores=NC, num_subcores=NS)
  f = functools.partial(
      pl.kernel,
      out_type=jax.ShapeDtypeStruct((CT, RT, 8, 128), jnp.float32),
      mesh=mesh,
      scratch_types=[
          pltpu.VMEM((C,), jnp.int32),
          pltpu.VMEM((C,), jnp.int32),
          pltpu.VMEM((C,), jnp.int32),
          pltpu.VMEM((C,), jnp.int32),
          pltpu.VMEM((C,), jnp.int32),
          pltpu.VMEM((C * AEP,), jnp.float32),
          pltpu.VMEM((C * AEP,), jnp.float32),
          pltpu.VMEM((NODE_V * NEP,), jnp.float32),
          pltpu.VMEM((TR, CTX_TC, 8, 128), jnp.float32),
          pltpu.VMEM((TR, ST_TC, 8, 128), jnp.float32),
          pltpu.VMEM((CT, 8, ACC * C), jnp.float32),
          pltpu.SemaphoreType.DMA,
          pltpu.SemaphoreType.DMA,
          pltpu.SemaphoreType.DMA,
      ],
      compiler_params=pltpu.CompilerParams(use_tc_tiling_on_sc=False,
                                           needs_layout_passes=False),
  )(_body)
  out4 = f(node_idx, act_p, st4, act_q, ctx4, rule_flat, token_flat,
           ntab_flat)
  return out4.transpose(0, 2, 1, 3).reshape(OUT_D, B).T


def kernel(current_node_type, previous_action, parent_state, parent_action,
           context, rule_embedding_table, token_embedding_table,
           node_embedding_table):
  act_p = previous_action.astype(jnp.int32).T.reshape(-1)
  act_q = parent_action.astype(jnp.int32).T.reshape(-1)
  ctx4 = context.reshape(B // 8, 8, CTX // 128, 128).transpose(0, 2, 1, 3)
  st4 = parent_state.reshape(B // 8, 8, ST // 128, 128).transpose(0, 2, 1, 3)
  rule_flat = jnp.pad(rule_embedding_table, ((0, 0), (0, AEP - AE))).reshape(-1)
  token_flat = jnp.pad(token_embedding_table, ((0, 0), (0, AEP - AE))).reshape(-1)
  ntab_flat = jnp.pad(node_embedding_table, ((0, 0), (0, NEP - NE))).reshape(-1)
  return _lstm_input(current_node_type.astype(jnp.int32), act_p, st4, act_q,
                     ctx4, rule_flat, token_flat, ntab_flat)


# speedup vs baseline: 1.1429x; 1.1409x over previous
import functools

import jax
import jax.numpy as jnp
from jax import lax
from jax.experimental import pallas as pl
from jax.experimental.pallas import tpu as pltpu
from jax.experimental.pallas import tpu_sc as plsc

B = 16384
AE = 50
AEP = 56
NE = 20
NEP = 24
NODE_V = 1000
CTX = 512
ST = 256
OUT_D = AE + CTX + AE + ST + NE
OFF_CTX = AE
OFF_PAR = AE + CTX
OFF_ST = OFF_PAR + AE
OFF_NODE = OFF_ST + ST
CT = OUT_D // 8
RT = B // 128

NC, NS, L = 2, 16, 16
NW = NC * NS
ROWS_PER_W = B // NW
C = 32
NCHUNK = ROWS_PER_W // C
TR = C // 8
CTX_TC = CTX // 128
ST_TC = ST // 128
ACC = 2


def _body(node_idx_hbm, act_p_hbm, st_hbm, act_q_hbm, ctx_hbm,
          rule_hbm, token_hbm, ntab_hbm, out_hbm,
          tp_v, vp_v, tq_v, vq_v, nidx_v, emb_p_v, emb_q_v, ntab_v,
          ctx_s, st_s, out_v, sem_in, sem_g, sem_n):
  wid = lax.axis_index("s") * NC + lax.axis_index("c")
  base_w = wid * ROWS_PER_W
  iota = lax.iota(jnp.int32, L)

  nt = pltpu.async_copy(ntab_hbm, ntab_v, sem_n)

  def chunk(g, carry):
    row0 = base_w + g * C
    rt0 = row0 // 128
    lane0 = row0 % 128
    acc0 = lax.rem(row0, ACC * C)

    i1 = pltpu.async_copy(act_p_hbm.at[pl.ds(row0, C)], tp_v, sem_in)
    i2 = pltpu.async_copy(act_p_hbm.at[pl.ds(B + row0, C)], vp_v, sem_in)
    i3 = pltpu.async_copy(act_q_hbm.at[pl.ds(row0, C)], tq_v, sem_in)
    i4 = pltpu.async_copy(act_q_hbm.at[pl.ds(B + row0, C)], vq_v, sem_in)
    i5 = pltpu.async_copy(node_idx_hbm.at[pl.ds(row0, C)], nidx_v, sem_in)
    d1 = pltpu.async_copy(ctx_hbm.at[pl.ds(row0 // 8, TR)], ctx_s, sem_in)
    d2 = pltpu.async_copy(st_hbm.at[pl.ds(row0 // 8, TR)], st_s, sem_in)
    i1.wait(); i2.wait(); i3.wait(); i4.wait(); i5.wait()

    def fetch_group(j, _):
      tp_vec = tp_v[pl.ds(j * L, L)]
      vp_vec = vp_v[pl.ds(j * L, L)]
      tq_vec = tq_v[pl.ds(j * L, L)]
      vq_vec = vq_v[pl.ds(j * L, L)]
      for l in range(L):
        i = j * L + l
        tp, vp = tp_vec[l], vp_vec[l]
        tq, vq = tq_vec[l], vq_vec[l]

        @pl.when(tp == 0)
        def _():
          pltpu.async_copy(rule_hbm.at[pl.ds(vp * AEP, AEP)],
                           emb_p_v.at[pl.ds(i * AEP, AEP)], sem_g)

        @pl.when(tp != 0)
        def _():
          pltpu.async_copy(token_hbm.at[pl.ds(vp * AEP, AEP)],
                           emb_p_v.at[pl.ds(i * AEP, AEP)], sem_g)

        @pl.when(tq == 0)
        def _():
          pltpu.async_copy(rule_hbm.at[pl.ds(vq * AEP, AEP)],
                           emb_q_v.at[pl.ds(i * AEP, AEP)], sem_g)

        @pl.when(tq != 0)
        def _():
          pltpu.async_copy(token_hbm.at[pl.ds(vq * AEP, AEP)],
                           emb_q_v.at[pl.ds(i * AEP, AEP)], sem_g)

      return 0

    lax.fori_loop(0, C // L, fetch_group, 0)

    d1.wait(); d2.wait()
    for seg_off, seg_tc, seg_s in ((OFF_CTX, CTX_TC, ctx_s),
                                   (OFF_ST, ST_TC, st_s)):
      for tc in range(seg_tc):
        for k in range(128 // L):
          c0 = seg_off + tc * 128 + k * L
          cvec = iota + c0
          cdiv = cvec // 8
          cmod = lax.rem(cvec, jnp.full((L,), 8, jnp.int32))
          for tr in range(TR):
            vs = [seg_s[tr, tc, sl, pl.ds(k * L, L)] for sl in range(8)]
            lanes = [iota * 0 + (acc0 + tr * 8 + sl) for sl in range(8)]
            for sl in range(8):
              plsc.store_scatter(out_v, [cdiv, cmod, lanes[sl]], vs[sl])

    def node_col(c, _):
      cc = c + OFF_NODE
      vs = [plsc.load_gather(ntab_v, [nidx_v[pl.ds(j * L, L)] * NEP + c])
            for j in range(C // L)]
      for j in range(C // L):
        out_v[cc // 8, lax.rem(cc, 8), pl.ds(acc0 + j * L, L)] = vs[j]
      return 0

    lax.fori_loop(0, NE, node_col, 0)

    pltpu.make_async_copy(rule_hbm.at[pl.ds(0, C * AEP)], emb_p_v,
                          sem_g).wait()
    pltpu.make_async_copy(rule_hbm.at[pl.ds(0, C * AEP)], emb_q_v,
                          sem_g).wait()

    def place_col(c, _):
      cq = c + OFF_PAR
      flats = [(iota + (j * L)) * AEP + c for j in range(C // L)]
      vps = [plsc.load_gather(emb_p_v, [f]) for f in flats]
      vqs = [plsc.load_gather(emb_q_v, [f]) for f in flats]
      for j in range(C // L):
        out_v[c // 8, lax.rem(c, 8), pl.ds(acc0 + j * L, L)] = vps[j]
        out_v[cq // 8, lax.rem(cq, 8), pl.ds(acc0 + j * L, L)] = vqs[j]
      return 0

    lax.fori_loop(0, AE, place_col, 0)

    @pl.when(lax.rem(g, ACC) == ACC - 1)
    def _():
      pltpu.sync_copy(out_v,
                      out_hbm.at[:, rt0, :, pl.ds(lane0 - (ACC - 1) * C,
                                                  ACC * C)])

    return carry

  nt.wait()
  lax.fori_loop(0, NCHUNK, chunk, 0)


@jax.jit
def _lstm_input(node_idx, act_p, st4, act_q, ctx4, rule_flat, token_flat,
                ntab_flat):
  mesh = plsc.VectorSubcoreMesh(core_axis_name="c", subcore_axis_name="s",
                                num_cores=NC, num_subcores=NS)
  f = functools.partial(
      pl.kernel,
      out_type=jax.ShapeDtypeStruct((CT, RT, 8, 128), jnp.float32),
      mesh=mesh,
      scratch_types=[
          pltpu.VMEM((C,), jnp.int32),
          pltpu.VMEM((C,), jnp.int32),
          pltpu.VMEM((C,), jnp.int32),
          pltpu.VMEM((C,), jnp.int32),
          pltpu.VMEM((C,), jnp.int32),
          pltpu.VMEM((C * AEP,), jnp.float32),
          pltpu.VMEM((C * AEP,), jnp.float32),
          pltpu.VMEM((NODE_V * NEP,), jnp.float32),
          pltpu.VMEM((TR, CTX_TC, 8, 128), jnp.float32),
          pltpu.VMEM((TR, ST_TC, 8, 128), jnp.float32),
          pltpu.VMEM((CT, 8, ACC * C), jnp.float32),
          pltpu.SemaphoreType.DMA,
          pltpu.SemaphoreType.DMA,
          pltpu.SemaphoreType.DMA,
      ],
      compiler_params=pltpu.CompilerParams(use_tc_tiling_on_sc=False,
                                           needs_layout_passes=False),
  )(_body)
  out4 = f(node_idx, act_p, st4, act_q, ctx4, rule_flat, token_flat,
           ntab_flat)
  return out4.transpose(0, 2, 1, 3).reshape(OUT_D, B).T


def kernel(current_node_type, previous_action, parent_state, parent_action,
           context, rule_embedding_table, token_embedding_table,
           node_embedding_table):
  act_p = previous_action.astype(jnp.int32).T.reshape(-1)
  act_q = parent_action.astype(jnp.int32).T.reshape(-1)
  ctx4 = context.reshape(B // 8, 8, CTX // 128, 128).transpose(0, 2, 1, 3)
  st4 = parent_state.reshape(B // 8, 8, ST // 128, 128).transpose(0, 2, 1, 3)
  rule_flat = jnp.pad(rule_embedding_table, ((0, 0), (0, AEP - AE))).reshape(-1)
  token_flat = jnp.pad(token_embedding_table, ((0, 0), (0, AEP - AE))).reshape(-1)
  ntab_flat = jnp.pad(node_embedding_table, ((0, 0), (0, NEP - NE))).reshape(-1)
  return _lstm_input(current_node_type.astype(jnp.int32), act_p, st4, act_q,
                     ctx4, rule_flat, token_flat, ntab_flat)


# speedup vs baseline: 1.2233x; 1.0704x over previous
import functools

import jax
import jax.numpy as jnp
from jax import lax
from jax.experimental import pallas as pl
from jax.experimental.pallas import tpu as pltpu
from jax.experimental.pallas import tpu_sc as plsc

B = 16384
AE = 50
AEP = 56
NE = 20
NEP = 24
NODE_V = 1000
CTX = 512
ST = 256
OUT_D = AE + CTX + AE + ST + NE
OFF_CTX = AE
OFF_PAR = AE + CTX
OFF_ST = OFF_PAR + AE
OFF_NODE = OFF_ST + ST

NC, NS, L = 2, 16, 16
NW = NC * NS
ROWS_PER_W = B // NW
C = 32
NCHUNK = ROWS_PER_W // C
TR = C // 8
CTX_TC = CTX // 128
ST_TC = ST // 128


def _body(node_idx_hbm, act_p_hbm, st_hbm, act_q_hbm, ctx_hbm,
          rule_hbm, token_hbm, ntab_hbm, out_hbm,
          tp_v, vp_v, tq_v, vq_v, nidx_v, emb_p_v, emb_q_v, ntab_v,
          ctx_s, st_s, out_v, sem_in, sem_g, sem_n):
  wid = lax.axis_index("s") * NC + lax.axis_index("c")
  base_w = wid * ROWS_PER_W
  iota = lax.iota(jnp.int32, L)

  nt = pltpu.async_copy(ntab_hbm, ntab_v, sem_n)

  def chunk(g, carry):
    row0 = base_w + g * C

    i1 = pltpu.async_copy(act_p_hbm.at[pl.ds(row0, C)], tp_v, sem_in)
    i2 = pltpu.async_copy(act_p_hbm.at[pl.ds(B + row0, C)], vp_v, sem_in)
    i3 = pltpu.async_copy(act_q_hbm.at[pl.ds(row0, C)], tq_v, sem_in)
    i4 = pltpu.async_copy(act_q_hbm.at[pl.ds(B + row0, C)], vq_v, sem_in)
    i5 = pltpu.async_copy(node_idx_hbm.at[pl.ds(row0, C)], nidx_v, sem_in)
    d1 = pltpu.async_copy(ctx_hbm.at[pl.ds(row0 // 8, TR)], ctx_s, sem_in)
    d2 = pltpu.async_copy(st_hbm.at[pl.ds(row0 // 8, TR)], st_s, sem_in)
    i1.wait(); i2.wait(); i3.wait(); i4.wait(); i5.wait()

    def fetch_group(j, _):
      tp_vec = tp_v[pl.ds(j * L, L)]
      vp_vec = vp_v[pl.ds(j * L, L)]
      tq_vec = tq_v[pl.ds(j * L, L)]
      vq_vec = vq_v[pl.ds(j * L, L)]
      for l in range(L):
        i = j * L + l
        tp, vp = tp_vec[l], vp_vec[l]
        tq, vq = tq_vec[l], vq_vec[l]

        @pl.when(tp == 0)
        def _():
          pltpu.async_copy(rule_hbm.at[pl.ds(vp * AEP, AEP)],
                           emb_p_v.at[pl.ds(i * AEP, AEP)], sem_g)

        @pl.when(tp != 0)
        def _():
          pltpu.async_copy(token_hbm.at[pl.ds(vp * AEP, AEP)],
                           emb_p_v.at[pl.ds(i * AEP, AEP)], sem_g)

        @pl.when(tq == 0)
        def _():
          pltpu.async_copy(rule_hbm.at[pl.ds(vq * AEP, AEP)],
                           emb_q_v.at[pl.ds(i * AEP, AEP)], sem_g)

        @pl.when(tq != 0)
        def _():
          pltpu.async_copy(token_hbm.at[pl.ds(vq * AEP, AEP)],
                           emb_q_v.at[pl.ds(i * AEP, AEP)], sem_g)

      return 0

    lax.fori_loop(0, C // L, fetch_group, 0)

    d1.wait(); d2.wait()
    for seg_off, seg_tc, seg_s in ((OFF_CTX, CTX_TC, ctx_s),
                                   (OFF_ST, ST_TC, st_s)):
      for tc in range(seg_tc):
        for tr in range(TR):
          for k in range(128 // L):
            vs = [seg_s[tr, tc, sl, pl.ds(k * L, L)] for sl in range(8)]
            for sl in range(8):
              r = tr * 8 + sl
              out_v[r, pl.ds(seg_off + tc * 128 + k * L, L)] = vs[sl]

    def node_col(c, _):
      vs = [plsc.load_gather(ntab_v, [nidx_v[pl.ds(j * L, L)] * NEP + c])
            for j in range(C // L)]
      for j in range(C // L):
        rows = iota + (j * L)
        plsc.store_scatter(out_v, [rows, iota * 0 + (c + OFF_NODE)], vs[j])
      return 0

    lax.fori_loop(0, NE, node_col, 0)

    pltpu.make_async_copy(rule_hbm.at[pl.ds(0, C * AEP)], emb_p_v,
                          sem_g).wait()
    pltpu.make_async_copy(rule_hbm.at[pl.ds(0, C * AEP)], emb_q_v,
                          sem_g).wait()

    def place_col(c, _):
      flats = [(iota + (j * L)) * AEP + c for j in range(C // L)]
      vps = [plsc.load_gather(emb_p_v, [f]) for f in flats]
      vqs = [plsc.load_gather(emb_q_v, [f]) for f in flats]
      for j in range(C // L):
        rows = iota + (j * L)
        plsc.store_scatter(out_v, [rows, iota * 0 + c], vps[j])
        plsc.store_scatter(out_v, [rows, iota * 0 + (c + OFF_PAR)], vqs[j])
      return 0

    lax.fori_loop(0, AE, place_col, 0)

    pltpu.sync_copy(out_v, out_hbm.at[pl.ds(row0, C), :])
    return carry

  nt.wait()
  lax.fori_loop(0, NCHUNK, chunk, 0)


@jax.jit
def _lstm_input(node_idx, act_p, st4, act_q, ctx4, rule_flat, token_flat,
                ntab_flat):
  mesh = plsc.VectorSubcoreMesh(core_axis_name="c", subcore_axis_name="s",
                                num_cores=NC, num_subcores=NS)
  f = functools.partial(
      pl.kernel,
      out_type=jax.ShapeDtypeStruct((B, OUT_D), jnp.float32),
      mesh=mesh,
      scratch_types=[
          pltpu.VMEM((C,), jnp.int32),
          pltpu.VMEM((C,), jnp.int32),
          pltpu.VMEM((C,), jnp.int32),
          pltpu.VMEM((C,), jnp.int32),
          pltpu.VMEM((C,), jnp.int32),
          pltpu.VMEM((C * AEP,), jnp.float32),
          pltpu.VMEM((C * AEP,), jnp.float32),
          pltpu.VMEM((NODE_V * NEP,), jnp.float32),
          pltpu.VMEM((TR, CTX_TC, 8, 128), jnp.float32),
          pltpu.VMEM((TR, ST_TC, 8, 128), jnp.float32),
          pltpu.VMEM((C, OUT_D), jnp.float32),
          pltpu.SemaphoreType.DMA,
          pltpu.SemaphoreType.DMA,
          pltpu.SemaphoreType.DMA,
      ],
      compiler_params=pltpu.CompilerParams(use_tc_tiling_on_sc=False,
                                           needs_layout_passes=False),
  )(_body)
  return f(node_idx, act_p, st4, act_q, ctx4, rule_flat, token_flat,
           ntab_flat)


def kernel(current_node_type, previous_action, parent_state, parent_action,
           context, rule_embedding_table, token_embedding_table,
           node_embedding_table):
  act_p = previous_action.astype(jnp.int32).T.reshape(-1)
  act_q = parent_action.astype(jnp.int32).T.reshape(-1)
  ctx4 = context.reshape(B // 8, 8, CTX // 128, 128).transpose(0, 2, 1, 3)
  st4 = parent_state.reshape(B // 8, 8, ST // 128, 128).transpose(0, 2, 1, 3)
  rule_flat = jnp.pad(rule_embedding_table, ((0, 0), (0, AEP - AE))).reshape(-1)
  token_flat = jnp.pad(token_embedding_table, ((0, 0), (0, AEP - AE))).reshape(-1)
  ntab_flat = jnp.pad(node_embedding_table, ((0, 0), (0, NEP - NE))).reshape(-1)
  return _lstm_input(current_node_type.astype(jnp.int32), act_p, st4, act_q,
                     ctx4, rule_flat, token_flat, ntab_flat)
